# Initial kernel scaffold; baseline (speedup 1.0000x reference)
import jax
import jax.numpy as jnp
from jax.experimental import pallas as pl


def kernel(x, edge_index, W1, b1, W2, b2):
    raise NotImplementedError("write your pallas kernel here")


# speedup vs baseline: 12.2280x; 12.2280x over previous
import functools

import jax
import jax.numpy as jnp
from jax import lax
from jax.experimental import pallas as pl
from jax.experimental.pallas import tpu as pltpu
from jax.experimental.pallas import tpu_sc as plsc

N = 10000
D = 128
E = 320000

NC = 2
NS = 16
NT = NC * NS
EC = 128
CPT = 79
E_PAD = NT * CPT * EC
N_PAD = 10240
RPT = N_PAD // NS
DEGW = 128

_f32 = jnp.float32


def _deg_body(dst_hbm, out_hbm, dst_buf, rows_buf, acc_sh):
    c = lax.axis_index("c")
    s = lax.axis_index("s")
    w = c * NS + s

    def fillval(val):
        def frow(i, carry):
            def fcol(j, carry2):
                rows_buf[i, pl.ds(j * 16, 16)] = jnp.full((16,), val, _f32)
                return carry2
            return lax.fori_loop(0, DEGW // 16, fcol, carry)
        lax.fori_loop(0, EC, frow, 0)

    fillval(0.0)

    def zslab(i, carry):
        pltpu.sync_copy(rows_buf, acc_sh.at[pl.ds(s * RPT + i * EC, EC)])
        return carry

    lax.fori_loop(0, RPT // EC, zslab, 0)
    plsc.subcore_barrier()

    fillval(1.0)
    pltpu.sync_copy(dst_hbm.at[w], dst_buf)

    def chunk(j, carry):
        pltpu.sync_copy(rows_buf, acc_sh.at[dst_buf.at[j]], add=True)
        return carry

    lax.fori_loop(0, CPT, chunk, 0)
    plsc.subcore_barrier()

    def cpout(i, carry):
        pltpu.sync_copy(acc_sh.at[pl.ds(s * RPT + i * EC, EC)], rows_buf)
        pltpu.sync_copy(rows_buf, out_hbm.at[c, pl.ds(s * RPT + i * EC, EC)])
        return carry

    lax.fori_loop(0, RPT // EC, cpout, 0)


_deg_kernel = pl.kernel(
    _deg_body,
    out_type=jax.ShapeDtypeStruct((NC, N_PAD, DEGW), _f32),
    mesh=plsc.VectorSubcoreMesh(core_axis_name="c", subcore_axis_name="s"),
    scratch_types=[
        pltpu.VMEM((CPT, EC), jnp.int32),
        pltpu.VMEM((EC, DEGW), _f32),
        pltpu.VMEM_SHARED((N_PAD, DEGW), _f32),
    ],
)


def _agg_body(src_hbm, dst_hbm, h_hbm, out_hbm,
              src_buf, dst_buf, rows_buf, sem, acc_sh):
    c = lax.axis_index("c")
    s = lax.axis_index("s")
    w = c * NS + s

    def zrow(i, carry):
        def zcol(j, carry2):
            rows_buf[i, pl.ds(j * 16, 16)] = jnp.zeros((16,), _f32)
            return carry2
        return lax.fori_loop(0, D // 16, zcol, carry)

    lax.fori_loop(0, EC, zrow, 0)

    def zslab(i, carry):
        pltpu.sync_copy(rows_buf, acc_sh.at[pl.ds(s * RPT + i * EC, EC)])
        return carry

    lax.fori_loop(0, RPT // EC, zslab, 0)
    plsc.subcore_barrier()

    pltpu.sync_copy(src_hbm.at[w], src_buf)
    pltpu.sync_copy(dst_hbm.at[w], dst_buf)

    def chunk(j, carry):
        pltpu.async_copy(h_hbm.at[src_buf.at[j]], rows_buf, sem).wait()
        pltpu.sync_copy(rows_buf, acc_sh.at[dst_buf.at[j]], add=True)
        return carry

    lax.fori_loop(0, CPT, chunk, 0)
    plsc.subcore_barrier()

    def cpout(i, carry):
        pltpu.sync_copy(acc_sh.at[pl.ds(s * RPT + i * EC, EC)], rows_buf)
        pltpu.sync_copy(rows_buf, out_hbm.at[c, pl.ds(s * RPT + i * EC, EC)])
        return carry

    lax.fori_loop(0, RPT // EC, cpout, 0)


_agg_kernel = pl.kernel(
    _agg_body,
    out_type=jax.ShapeDtypeStruct((NC, N_PAD, D), _f32),
    mesh=plsc.VectorSubcoreMesh(core_axis_name="c", subcore_axis_name="s"),
    scratch_types=[
        pltpu.VMEM((CPT, EC), jnp.int32),
        pltpu.VMEM((CPT, EC), jnp.int32),
        pltpu.VMEM((EC, D), _f32),
        pltpu.SemaphoreType.DMA,
        pltpu.VMEM_SHARED((N_PAD, D), _f32),
    ],
)


BR = 640


def _dinv(da_ref, db_ref):
    deg = da_ref[:, 0:1] + db_ref[:, 0:1] + 1.0
    return lax.rsqrt(deg)


def _mm_scale_body(x_ref, w_ref, da_ref, db_ref, o_ref):
    h = jnp.dot(x_ref[...], w_ref[...], preferred_element_type=_f32)
    o_ref[...] = h * _dinv(da_ref, db_ref)


def _mid_body(aa_ref, ab_ref, hp_ref, da_ref, db_ref, b_ref, w_ref, o_ref):
    dinv = _dinv(da_ref, db_ref)
    pre = (aa_ref[...] + ab_ref[...] + hp_ref[...]) * dinv + b_ref[...]
    out1 = jnp.maximum(pre, 0.0)
    h2 = jnp.dot(out1, w_ref[...], preferred_element_type=_f32)
    o_ref[...] = h2 * dinv


def _fin_body(aa_ref, ab_ref, hp_ref, da_ref, db_ref, b_ref, o_ref):
    dinv = _dinv(da_ref, db_ref)
    o_ref[...] = (aa_ref[...] + ab_ref[...] + hp_ref[...]) * dinv + b_ref[...]


_row_spec = pl.BlockSpec((BR, D), lambda i: (i, 0))
_deg_spec = pl.BlockSpec((BR, DEGW), lambda i: (i, 0))
_w_spec = pl.BlockSpec((D, D), lambda i: (0, 0))
_b_spec = pl.BlockSpec((1, D), lambda i: (0, 0))
_out_struct = jax.ShapeDtypeStruct((N_PAD, D), _f32)
_grid = N_PAD // BR

_mm_scale = pl.pallas_call(
    _mm_scale_body,
    grid=_grid,
    in_specs=[_row_spec, _w_spec, _deg_spec, _deg_spec],
    out_specs=_row_spec,
    out_shape=_out_struct,
)

_mid = pl.pallas_call(
    _mid_body,
    grid=_grid,
    in_specs=[_row_spec, _row_spec, _row_spec, _deg_spec, _deg_spec,
              _b_spec, _w_spec],
    out_specs=_row_spec,
    out_shape=_out_struct,
)

_fin = pl.pallas_call(
    _fin_body,
    grid=_grid,
    in_specs=[_row_spec, _row_spec, _row_spec, _deg_spec, _deg_spec, _b_spec],
    out_specs=_row_spec,
    out_shape=_out_struct,
)


@jax.jit
def kernel(x, edge_index, W1, b1, W2, b2):
    ei = edge_index.astype(jnp.int32)
    pad = jnp.full((E_PAD - E,), N, jnp.int32)
    src_t = jnp.concatenate([ei[0], pad]).reshape(NT, CPT, EC)
    dst_t = jnp.concatenate([ei[1], pad]).reshape(NT, CPT, EC)
    x_pad = jnp.concatenate([x, jnp.zeros((N_PAD - N, D), _f32)])

    deg2 = _deg_kernel(dst_t)
    dega, degb = deg2[0], deg2[1]

    h1p = _mm_scale(x_pad, W1, dega, degb)
    acc1 = _agg_kernel(src_t, dst_t, h1p)
    h2p = _mid(acc1[0], acc1[1], h1p, dega, degb, b1.reshape(1, D), W2)
    acc2 = _agg_kernel(src_t, dst_t, h2p)
    out = _fin(acc2[0], acc2[1], h2p, dega, degb, b2.reshape(1, D))
    return out[:N]
